# baseline (speedup 1.0000x reference)
import functools

import jax
import jax.numpy as jnp
from jax import lax
from jax.experimental import pallas as pl
from jax.experimental.pallas import tpu as pltpu
from jax.experimental.pallas import tpu_sc as plsc

B, C, H, W = 32, 192, 56, 56
S = H * W
G = C // 2


def kernel(x1, x2):
    x1r = x1.reshape(B, C, S)
    x2r = x2.reshape(B, C, S)

    mesh = plsc.VectorSubcoreMesh(core_axis_name="c", subcore_axis_name="s")

    @functools.partial(
        pl.kernel,
        out_type=[
            jax.ShapeDtypeStruct((B, G, 2 * S), jnp.float32),
            jax.ShapeDtypeStruct((B, G, 2 * S), jnp.float32),
        ],
        mesh=mesh,
        scratch_types=[pltpu.SemaphoreType.DMA],
        compiler_params=pltpu.CompilerParams(use_tc_tiling_on_sc=False),
    )
    def shuffle(x1_hbm, x2_hbm, o1_hbm, o2_hbm, sem):
        b = lax.axis_index("s") * 2 + lax.axis_index("c")
        c1 = pltpu.async_copy(
            x1_hbm.at[b, pl.ds(0, G), :], o1_hbm.at[b, :, pl.ds(0, S)], sem)
        c2 = pltpu.async_copy(
            x2_hbm.at[b, pl.ds(0, G), :], o1_hbm.at[b, :, pl.ds(S, S)], sem)
        c3 = pltpu.async_copy(
            x1_hbm.at[b, pl.ds(G, G), :], o2_hbm.at[b, :, pl.ds(0, S)], sem)
        c4 = pltpu.async_copy(
            x2_hbm.at[b, pl.ds(G, G), :], o2_hbm.at[b, :, pl.ds(S, S)], sem)
        c1.wait()
        c2.wait()
        c3.wait()
        c4.wait()

    o1, o2 = shuffle(x1r, x2r)
    return o1.reshape(B, C, H, W), o2.reshape(B, C, H, W)


# speedup vs baseline: 4.6735x; 4.6735x over previous
import functools

import jax
import jax.numpy as jnp
from jax import lax
from jax.experimental import pallas as pl
from jax.experimental.pallas import tpu as pltpu
from jax.experimental.pallas import tpu_sc as plsc

B, C, H, W = 32, 192, 56, 56
S = H * W
G = C // 2
K = 8
NSTEP = G // K


def kernel(x1, x2):
    x1r = x1.reshape(B, C, S)
    x2r = x2.reshape(B, C, S)

    mesh = plsc.VectorSubcoreMesh(core_axis_name="c", subcore_axis_name="s")

    @functools.partial(
        pl.kernel,
        out_type=[
            jax.ShapeDtypeStruct((B, G, 2 * S), jnp.float32),
            jax.ShapeDtypeStruct((B, G, 2 * S), jnp.float32),
        ],
        mesh=mesh,
        scratch_types=[
            pltpu.VMEM((2, K, 2 * S), jnp.float32),
            pltpu.SemaphoreType.DMA,
            pltpu.SemaphoreType.DMA,
            pltpu.SemaphoreType.DMA,
            pltpu.SemaphoreType.DMA,
        ],
        compiler_params=pltpu.CompilerParams(use_tc_tiling_on_sc=False),
    )
    def shuffle(x1_hbm, x2_hbm, o1_hbm, o2_hbm, buf, in0, in1, out0, out1):
        b = lax.axis_index("s") * 2 + lax.axis_index("c")
        in_sems = (in0, in1)
        out_sems = (out0, out1)
        out_h = [None, None]
        for t in range(2 * NSTEP):
            p = t % 2
            if t < NSTEP:
                rows = pl.ds(t * K, K)
                dst = o1_hbm.at[b, rows, :]
            else:
                rows = pl.ds(G + (t - NSTEP) * K, K)
                dst = o2_hbm.at[b, pl.ds((t - NSTEP) * K, K), :]
            if out_h[p] is not None:
                out_h[p].wait()
            h1 = pltpu.async_copy(
                x1_hbm.at[b, rows, :], buf.at[p, :, pl.ds(0, S)], in_sems[p])
            h2 = pltpu.async_copy(
                x2_hbm.at[b, rows, :], buf.at[p, :, pl.ds(S, S)], in_sems[p])
            h1.wait()
            h2.wait()
            out_h[p] = pltpu.async_copy(buf.at[p], dst, out_sems[p])
        out_h[0].wait()
        out_h[1].wait()

    o1, o2 = shuffle(x1r, x2r)
    return o1.reshape(B, C, H, W), o2.reshape(B, C, H, W)


# speedup vs baseline: 4.6810x; 1.0016x over previous
import functools

import jax
import jax.numpy as jnp
from jax import lax
from jax.experimental import pallas as pl
from jax.experimental.pallas import tpu as pltpu
from jax.experimental.pallas import tpu_sc as plsc

B, C, H, W = 32, 192, 56, 56
S = H * W
G = C // 2
K = 8
NSTEP = G // K


def kernel(x1, x2):
    x1r = x1.reshape(B, C, S)
    x2r = x2.reshape(B, C, S)

    mesh = plsc.VectorSubcoreMesh(core_axis_name="c", subcore_axis_name="s")

    @functools.partial(
        pl.kernel,
        out_type=[
            jax.ShapeDtypeStruct((B, G, 2 * S), jnp.float32),
            jax.ShapeDtypeStruct((B, G, 2 * S), jnp.float32),
        ],
        mesh=mesh,
        scratch_types=[
            pltpu.VMEM((2, K, 2 * S), jnp.float32),
            pltpu.SemaphoreType.DMA,
            pltpu.SemaphoreType.DMA,
            pltpu.SemaphoreType.DMA,
            pltpu.SemaphoreType.DMA,
        ],
        compiler_params=pltpu.CompilerParams(use_tc_tiling_on_sc=False),
    )
    def shuffle(x1_hbm, x2_hbm, o1_hbm, o2_hbm, buf, in0, in1, out0, out1):
        b = lax.axis_index("s") * 2 + lax.axis_index("c")
        in_sems = (in0, in1)
        out_sems = (out0, out1)
        T = 2 * NSTEP

        def fire_in(t):
            p = t % 2
            if t < NSTEP:
                rows = pl.ds(t * K, K)
            else:
                rows = pl.ds(G + (t - NSTEP) * K, K)
            h1 = pltpu.async_copy(
                x1_hbm.at[b, rows, :], buf.at[p, :, pl.ds(0, S)], in_sems[p])
            h2 = pltpu.async_copy(
                x2_hbm.at[b, rows, :], buf.at[p, :, pl.ds(S, S)], in_sems[p])
            return (h1, h2)

        def fire_out(t):
            p = t % 2
            if t < NSTEP:
                dst = o1_hbm.at[b, pl.ds(t * K, K), :]
            else:
                dst = o2_hbm.at[b, pl.ds((t - NSTEP) * K, K), :]
            return pltpu.async_copy(buf.at[p], dst, out_sems[p])

        in_h = [None, None]
        out_h = [None, None]
        in_h[0] = fire_in(0)
        for t in range(T):
            p = t % 2
            q = (t + 1) % 2
            if t + 1 < T:
                if out_h[q] is not None:
                    out_h[q].wait()
                    out_h[q] = None
                in_h[q] = fire_in(t + 1)
            in_h[p][0].wait()
            in_h[p][1].wait()
            out_h[p] = fire_out(t)
        out_h[(T - 1) % 2].wait()
        if out_h[T % 2] is not None:
            out_h[T % 2].wait()

    o1, o2 = shuffle(x1r, x2r)
    return o1.reshape(B, C, H, W), o2.reshape(B, C, H, W)


# speedup vs baseline: 7.3676x; 1.5739x over previous
import functools

import jax
import jax.numpy as jnp
from jax import lax
from jax.experimental import pallas as pl
from jax.experimental.pallas import tpu as pltpu
from jax.experimental.pallas import tpu_sc as plsc

B, C, H, W = 32, 192, 56, 56
G = C // 2
K = 4
NSTEP = G // K


def kernel(x1, x2):
    mesh = plsc.VectorSubcoreMesh(core_axis_name="c", subcore_axis_name="s")

    @functools.partial(
        pl.kernel,
        out_type=[
            jax.ShapeDtypeStruct((B, G, 2, H, W), jnp.float32),
            jax.ShapeDtypeStruct((B, G, 2, H, W), jnp.float32),
        ],
        mesh=mesh,
        scratch_types=[
            pltpu.VMEM((K, 2, H, W), jnp.float32),
            pltpu.VMEM((K, 2, H, W), jnp.float32),
            pltpu.SemaphoreType.DMA,
            pltpu.SemaphoreType.DMA,
            pltpu.SemaphoreType.DMA,
            pltpu.SemaphoreType.DMA,
        ],
    )
    def shuffle(x1_hbm, x2_hbm, o1_hbm, o2_hbm, buf0, buf1, in0, in1,
                out0, out1):
        b = lax.axis_index("s") * 2 + lax.axis_index("c")
        bufs = (buf0, buf1)
        in_sems = (in0, in1)
        out_sems = (out0, out1)
        T = 2 * NSTEP

        def fire_in(t):
            p = t % 2
            if t < NSTEP:
                rows = pl.ds(t * K, K)
            else:
                rows = pl.ds(G + (t - NSTEP) * K, K)
            h1 = pltpu.async_copy(
                x1_hbm.at[b, rows, :, :], bufs[p].at[:, 0, :, :], in_sems[p])
            h2 = pltpu.async_copy(
                x2_hbm.at[b, rows, :, :], bufs[p].at[:, 1, :, :], in_sems[p])
            return (h1, h2)

        def fire_out(t):
            p = t % 2
            if t < NSTEP:
                dst = o1_hbm.at[b, pl.ds(t * K, K), :, :, :]
            else:
                dst = o2_hbm.at[b, pl.ds((t - NSTEP) * K, K), :, :, :]
            return pltpu.async_copy(bufs[p], dst, out_sems[p])

        in_h = [None, None]
        out_h = [None, None]
        in_h[0] = fire_in(0)
        for t in range(T):
            p = t % 2
            q = (t + 1) % 2
            if t + 1 < T:
                if out_h[q] is not None:
                    out_h[q].wait()
                    out_h[q] = None
                in_h[q] = fire_in(t + 1)
            in_h[p][0].wait()
            in_h[p][1].wait()
            out_h[p] = fire_out(t)
        out_h[(T - 1) % 2].wait()
        if out_h[T % 2] is not None:
            out_h[T % 2].wait()

    o1, o2 = shuffle(x1, x2)
    return o1.reshape(B, C, H, W), o2.reshape(B, C, H, W)


# speedup vs baseline: 20.2634x; 2.7503x over previous
import functools

import jax
import jax.numpy as jnp
from jax import lax
from jax.experimental import pallas as pl
from jax.experimental.pallas import tpu as pltpu
from jax.experimental.pallas import tpu_sc as plsc

B, C, H, W = 32, 192, 56, 56
G = C // 2
ROWS = B * H
NW = 32
RPW = ROWS // NW
NL = 16
NV = C // NL


def kernel(x1, x2):
    xt1 = jnp.transpose(x1, (0, 2, 3, 1)).reshape(ROWS, W, C)
    xt2 = jnp.transpose(x2, (0, 2, 3, 1)).reshape(ROWS, W, C)

    mesh = plsc.VectorSubcoreMesh(core_axis_name="c", subcore_axis_name="s")

    @functools.partial(
        pl.kernel,
        out_type=[
            jax.ShapeDtypeStruct((ROWS, W, C), jnp.float32),
            jax.ShapeDtypeStruct((ROWS, W, C), jnp.float32),
        ],
        mesh=mesh,
        scratch_types=[
            pltpu.VMEM((2, 2, W, C), jnp.float32),
            pltpu.VMEM((2, 2, W, C), jnp.float32),
            pltpu.SemaphoreType.DMA,
            pltpu.SemaphoreType.DMA,
            pltpu.SemaphoreType.DMA,
            pltpu.SemaphoreType.DMA,
        ],
        compiler_params=pltpu.CompilerParams(
            use_tc_tiling_on_sc=True, needs_layout_passes=False),
    )
    def shuffle(x1_hbm, x2_hbm, o1_hbm, o2_hbm, ibuf, obuf,
                in0, in1, out0, out1):
        wid = lax.axis_index("s") * 2 + lax.axis_index("c")
        base = wid * RPW
        in_sems = (in0, in1)
        out_sems = (out0, out1)

        lane = lax.iota(jnp.int32, NL)
        two = jnp.full((NL,), 2, jnp.int32)
        alt = lax.rem(lane, two)
        flr = lax.div(lane, two)
        cidx = [[lax.add(flr, jnp.full((NL,), G * o + (NL // 2) * v,
                                       jnp.int32))
                 for v in range(NV)]
                for o in range(2)]

        def fire_in(t, p):
            r = base + t
            pltpu.async_copy(x1_hbm.at[r], ibuf.at[p, 0], in_sems[p])
            pltpu.async_copy(x2_hbm.at[r], ibuf.at[p, 1], in_sems[p])

        def wait_in(t, p):
            r = base + t
            pltpu.make_async_copy(
                x1_hbm.at[r], ibuf.at[p, 0], in_sems[p]).wait()
            pltpu.make_async_copy(
                x2_hbm.at[r], ibuf.at[p, 1], in_sems[p]).wait()

        def fire_out(t, p):
            r = base + t
            pltpu.async_copy(obuf.at[p, 0], o1_hbm.at[r], out_sems[p])
            pltpu.async_copy(obuf.at[p, 1], o2_hbm.at[r], out_sems[p])

        def wait_out(t, p):
            r = base + t
            pltpu.make_async_copy(
                obuf.at[p, 0], o1_hbm.at[r], out_sems[p]).wait()
            pltpu.make_async_copy(
                obuf.at[p, 1], o2_hbm.at[r], out_sems[p]).wait()

        def compute(p):
            src = ibuf.at[p]

            def wbody(w, carry):
                wv = jnp.full((NL,), w, jnp.int32)
                for o in range(2):
                    for v in range(NV):
                        val = plsc.load_gather(src, [alt, wv, cidx[o][v]])
                        obuf[p, o, w, pl.ds(NL * v, NL)] = val
                return carry

            lax.fori_loop(0, W, wbody, 0)

        fire_in(0, 0)
        fire_in(1, 1)

        def body(t2, carry):
            for p in (0, 1):
                t = 2 * t2 + p
                wait_in(t, p)

                @pl.when(t2 > 0)
                def _():
                    wait_out(t - 2, p)

                compute(p)
                fire_out(t, p)

                @pl.when(t2 < RPW // 2 - 1)
                def _():
                    fire_in(t + 2, p)
            return carry

        lax.fori_loop(0, RPW // 2, body, 0)
        wait_out(RPW - 2, 0)
        wait_out(RPW - 1, 1)

    o1, o2 = shuffle(xt1, xt2)
    o1 = jnp.transpose(o1.reshape(B, H, W, C), (0, 3, 1, 2))
    o2 = jnp.transpose(o2.reshape(B, H, W, C), (0, 3, 1, 2))
    return o1, o2


# speedup vs baseline: 39.9007x; 1.9691x over previous
import functools

import jax
import jax.numpy as jnp
from jax import lax
from jax.experimental import pallas as pl
from jax.experimental.pallas import tpu as pltpu
from jax.experimental.pallas import tpu_sc as plsc

B, C, H, W = 32, 192, 56, 56
G = C // 2
ROWS = B * H
NW = 32
RPW = ROWS // NW
NL = 16
NV = C // NL


def kernel(x1, x2):
    xt1 = jnp.transpose(x1, (0, 2, 3, 1)).reshape(ROWS, W, C)
    xt2 = jnp.transpose(x2, (0, 2, 3, 1)).reshape(ROWS, W, C)

    mesh = plsc.VectorSubcoreMesh(core_axis_name="c", subcore_axis_name="s")

    @functools.partial(
        pl.kernel,
        out_type=[
            jax.ShapeDtypeStruct((ROWS, W, C), jnp.float32),
            jax.ShapeDtypeStruct((ROWS, W, C), jnp.float32),
        ],
        mesh=mesh,
        scratch_types=[
            pltpu.VMEM((2, 2, W, C), jnp.float32),
            pltpu.VMEM((2, 2, W, C), jnp.float32),
            pltpu.SemaphoreType.DMA,
            pltpu.SemaphoreType.DMA,
            pltpu.SemaphoreType.DMA,
            pltpu.SemaphoreType.DMA,
        ],
        compiler_params=pltpu.CompilerParams(
            use_tc_tiling_on_sc=True, needs_layout_passes=False),
    )
    def shuffle(x1_hbm, x2_hbm, o1_hbm, o2_hbm, ibuf, obuf,
                in0, in1, out0, out1):
        wid = lax.axis_index("s") * 2 + lax.axis_index("c")
        base = wid * RPW
        in_sems = (in0, in1)
        out_sems = (out0, out1)

        lane = lax.iota(jnp.int32, NL)
        two = jnp.full((NL,), 2, jnp.int32)
        alt = lax.rem(lane, two)
        flr = lax.div(lane, two)
        cidx = [[lax.add(flr, jnp.full((NL,), G * o + (NL // 2) * v,
                                       jnp.int32))
                 for v in range(NV)]
                for o in range(2)]

        def fire_in(t, p):
            r = base + t
            pltpu.async_copy(x1_hbm.at[r], ibuf.at[p, 0], in_sems[p])
            pltpu.async_copy(x2_hbm.at[r], ibuf.at[p, 1], in_sems[p])

        def wait_in(t, p):
            r = base + t
            pltpu.make_async_copy(
                x1_hbm.at[r], ibuf.at[p, 0], in_sems[p]).wait()
            pltpu.make_async_copy(
                x2_hbm.at[r], ibuf.at[p, 1], in_sems[p]).wait()

        def fire_out(t, p):
            r = base + t
            pltpu.async_copy(obuf.at[p, 0], o1_hbm.at[r], out_sems[p])
            pltpu.async_copy(obuf.at[p, 1], o2_hbm.at[r], out_sems[p])

        def wait_out(t, p):
            r = base + t
            pltpu.make_async_copy(
                obuf.at[p, 0], o1_hbm.at[r], out_sems[p]).wait()
            pltpu.make_async_copy(
                obuf.at[p, 1], o2_hbm.at[r], out_sems[p]).wait()

        def compute(p):
            src = ibuf.at[p]

            @plsc.parallel_loop(0, W, 1, unroll=2)
            def wbody(w):
                wv = jnp.full((NL,), w, jnp.int32)
                vals = [plsc.load_gather(src, [alt, wv, cidx[o][v]])
                        for o in range(2) for v in range(NV)]
                for o in range(2):
                    for v in range(NV):
                        obuf[p, o, w, pl.ds(NL * v, NL)] = vals[o * NV + v]

        fire_in(0, 0)
        fire_in(1, 1)

        def body(t2, carry):
            for p in (0, 1):
                t = 2 * t2 + p
                wait_in(t, p)

                @pl.when(t2 > 0)
                def _():
                    wait_out(t - 2, p)

                compute(p)
                fire_out(t, p)

                @pl.when(t2 < RPW // 2 - 1)
                def _():
                    fire_in(t + 2, p)
            return carry

        lax.fori_loop(0, RPW // 2, body, 0)
        wait_out(RPW - 2, 0)
        wait_out(RPW - 1, 1)

    o1, o2 = shuffle(xt1, xt2)
    o1 = jnp.transpose(o1.reshape(B, H, W, C), (0, 3, 1, 2))
    o2 = jnp.transpose(o2.reshape(B, H, W, C), (0, 3, 1, 2))
    return o1, o2


# speedup vs baseline: 39.9796x; 1.0020x over previous
import functools

import jax
import jax.numpy as jnp
from jax import lax
from jax.experimental import pallas as pl
from jax.experimental.pallas import tpu as pltpu
from jax.experimental.pallas import tpu_sc as plsc

B, C, H, W = 32, 192, 56, 56
G = C // 2
ROWS = B * H
NW = 32
RPW = ROWS // NW
NL = 16
NV = C // NL


def kernel(x1, x2):
    xt1 = jnp.transpose(x1, (0, 2, 3, 1)).reshape(ROWS, W, C)
    xt2 = jnp.transpose(x2, (0, 2, 3, 1)).reshape(ROWS, W, C)

    mesh = plsc.VectorSubcoreMesh(core_axis_name="c", subcore_axis_name="s")

    @functools.partial(
        pl.kernel,
        out_type=[
            jax.ShapeDtypeStruct((ROWS, W, C), jnp.float32),
            jax.ShapeDtypeStruct((ROWS, W, C), jnp.float32),
        ],
        mesh=mesh,
        scratch_types=[
            pltpu.VMEM((2, 2, W, C), jnp.float32),
            pltpu.VMEM((2, 2, W, C), jnp.float32),
            pltpu.SemaphoreType.DMA,
            pltpu.SemaphoreType.DMA,
            pltpu.SemaphoreType.DMA,
            pltpu.SemaphoreType.DMA,
        ],
        compiler_params=pltpu.CompilerParams(
            use_tc_tiling_on_sc=True, needs_layout_passes=False),
    )
    def shuffle(x1_hbm, x2_hbm, o1_hbm, o2_hbm, ibuf, obuf,
                in0, in1, out0, out1):
        wid = lax.axis_index("s") * 2 + lax.axis_index("c")
        base = wid * RPW
        in_sems = (in0, in1)
        out_sems = (out0, out1)

        lane = lax.iota(jnp.int32, NL)
        two = jnp.full((NL,), 2, jnp.int32)
        alt = lax.rem(lane, two)
        flr = lax.div(lane, two)
        cidx = [[lax.add(flr, jnp.full((NL,), G * o + (NL // 2) * v,
                                       jnp.int32))
                 for v in range(NV)]
                for o in range(2)]

        def fire_in(t, p):
            r = base + t
            pltpu.async_copy(x1_hbm.at[r], ibuf.at[p, 0], in_sems[p])
            pltpu.async_copy(x2_hbm.at[r], ibuf.at[p, 1], in_sems[p])

        def wait_in(t, p):
            r = base + t
            pltpu.make_async_copy(
                x1_hbm.at[r], ibuf.at[p, 0], in_sems[p]).wait()
            pltpu.make_async_copy(
                x2_hbm.at[r], ibuf.at[p, 1], in_sems[p]).wait()

        def fire_out(t, p):
            r = base + t
            pltpu.async_copy(obuf.at[p, 0], o1_hbm.at[r], out_sems[p])
            pltpu.async_copy(obuf.at[p, 1], o2_hbm.at[r], out_sems[p])

        def wait_out(t, p):
            r = base + t
            pltpu.make_async_copy(
                obuf.at[p, 0], o1_hbm.at[r], out_sems[p]).wait()
            pltpu.make_async_copy(
                obuf.at[p, 1], o2_hbm.at[r], out_sems[p]).wait()

        def compute(p):
            src = ibuf.at[p]

            @plsc.parallel_loop(0, W, 1, unroll=4)
            def wbody(w):
                wv = jnp.full((NL,), w, jnp.int32)
                vals = [plsc.load_gather(src, [alt, wv, cidx[o][v]])
                        for o in range(2) for v in range(NV)]
                for o in range(2):
                    for v in range(NV):
                        obuf[p, o, w, pl.ds(NL * v, NL)] = vals[o * NV + v]

        fire_in(0, 0)
        fire_in(1, 1)

        def body(t2, carry):
            for p in (0, 1):
                t = 2 * t2 + p
                wait_in(t, p)

                @pl.when(t2 > 0)
                def _():
                    wait_out(t - 2, p)

                compute(p)
                fire_out(t, p)

                @pl.when(t2 < RPW // 2 - 1)
                def _():
                    fire_in(t + 2, p)
            return carry

        lax.fori_loop(0, RPW // 2, body, 0)
        wait_out(RPW - 2, 0)
        wait_out(RPW - 1, 1)

    o1, o2 = shuffle(xt1, xt2)
    o1 = jnp.transpose(o1.reshape(B, H, W, C), (0, 3, 1, 2))
    o2 = jnp.transpose(o2.reshape(B, H, W, C), (0, 3, 1, 2))
    return o1, o2


# speedup vs baseline: 40.3188x; 1.0085x over previous
import functools

import jax
import jax.numpy as jnp
from jax import lax
from jax.experimental import pallas as pl
from jax.experimental.pallas import tpu as pltpu
from jax.experimental.pallas import tpu_sc as plsc

B, C, H, W = 32, 192, 56, 56
G = C // 2
ROWS = B * H
NW = 32
RPW = ROWS // NW
NL = 16
NV = C // NL


def kernel(x1, x2):
    xt1 = jnp.transpose(x1, (0, 2, 3, 1)).reshape(ROWS, W, C)
    xt2 = jnp.transpose(x2, (0, 2, 3, 1)).reshape(ROWS, W, C)

    mesh = plsc.VectorSubcoreMesh(core_axis_name="c", subcore_axis_name="s")

    @functools.partial(
        pl.kernel,
        out_type=[
            jax.ShapeDtypeStruct((ROWS, W, C), jnp.float32),
            jax.ShapeDtypeStruct((ROWS, W, C), jnp.float32),
        ],
        mesh=mesh,
        scratch_types=[
            pltpu.VMEM((2, 2, W, C), jnp.float32),
            pltpu.VMEM((2, 2, W, C), jnp.float32),
            pltpu.SemaphoreType.DMA,
            pltpu.SemaphoreType.DMA,
            pltpu.SemaphoreType.DMA,
            pltpu.SemaphoreType.DMA,
        ],
        compiler_params=pltpu.CompilerParams(
            use_tc_tiling_on_sc=True, needs_layout_passes=False),
    )
    def shuffle(x1_hbm, x2_hbm, o1_hbm, o2_hbm, ibuf, obuf,
                in0, in1, out0, out1):
        wid = lax.axis_index("s") * 2 + lax.axis_index("c")
        base = wid * RPW
        in_sems = (in0, in1)
        out_sems = (out0, out1)

        lane = lax.iota(jnp.int32, NL)
        two = jnp.full((NL,), 2, jnp.int32)
        alt = lax.rem(lane, two)
        flr = lax.div(lane, two)
        cidx = [[lax.add(flr, jnp.full((NL,), G * o + (NL // 2) * v,
                                       jnp.int32))
                 for v in range(NV)]
                for o in range(2)]

        def fire_in(t, p):
            r = base + t
            pltpu.async_copy(x1_hbm.at[r], ibuf.at[p, 0], in_sems[p])
            pltpu.async_copy(x2_hbm.at[r], ibuf.at[p, 1], in_sems[p])

        def wait_in(t, p):
            r = base + t
            pltpu.make_async_copy(
                x1_hbm.at[r], ibuf.at[p, 0], in_sems[p]).wait()
            pltpu.make_async_copy(
                x2_hbm.at[r], ibuf.at[p, 1], in_sems[p]).wait()

        def fire_out(t, p, o):
            r = base + t
            dst = o1_hbm if o == 0 else o2_hbm
            pltpu.async_copy(obuf.at[p, o], dst.at[r], out_sems[p])

        def wait_out(t, p):
            r = base + t
            pltpu.make_async_copy(
                obuf.at[p, 0], o1_hbm.at[r], out_sems[p]).wait()
            pltpu.make_async_copy(
                obuf.at[p, 1], o2_hbm.at[r], out_sems[p]).wait()

        def compute(p, o):
            src = ibuf.at[p]

            @plsc.parallel_loop(0, W, 1, unroll=4)
            def wbody(w):
                wv = jnp.full((NL,), w, jnp.int32)
                vals = [plsc.load_gather(src, [alt, wv, cidx[o][v]])
                        for v in range(NV)]
                for v in range(NV):
                    obuf[p, o, w, pl.ds(NL * v, NL)] = vals[v]

        fire_in(0, 0)
        fire_in(1, 1)

        def body(t2, carry):
            for p in (0, 1):
                t = 2 * t2 + p
                wait_in(t, p)

                @pl.when(t2 > 0)
                def _():
                    wait_out(t - 2, p)

                compute(p, 0)
                fire_out(t, p, 0)
                compute(p, 1)
                fire_out(t, p, 1)

                @pl.when(t2 < RPW // 2 - 1)
                def _():
                    fire_in(t + 2, p)
            return carry

        lax.fori_loop(0, RPW // 2, body, 0)
        wait_out(RPW - 2, 0)
        wait_out(RPW - 1, 1)

    o1, o2 = shuffle(xt1, xt2)
    o1 = jnp.transpose(o1.reshape(B, H, W, C), (0, 3, 1, 2))
    o2 = jnp.transpose(o2.reshape(B, H, W, C), (0, 3, 1, 2))
    return o1, o2
